# Initial kernel scaffold; baseline (speedup 1.0000x reference)
import jax
import jax.numpy as jnp
from jax.experimental import pallas as pl


def kernel(x, edge_index, params):
    raise NotImplementedError("write your pallas kernel here")


# speedup vs baseline: 7.5593x; 7.5593x over previous
import dataclasses
import jax
import jax.numpy as jnp
from jax import lax
from jax.experimental import pallas as pl
from jax.experimental.pallas import tpu as pltpu
from jax.experimental.pallas import tpu_sc as plsc

NN = 10000
EE = 320000
SLOPE_GAT = 0.2
NTILES = 32
EPT = EE // NTILES
CHB = 4000

_SC_CP = pltpu.CompilerParams()
if "needs_layout_passes" in pltpu.CompilerParams.__dataclass_fields__:
    _SC_CP = dataclasses.replace(_SC_CP, needs_layout_passes=False)
_TC_CP = pltpu.CompilerParams(vmem_limit_bytes=100 * 1024 * 1024)

_mesh_cache = []


def _mesh():
    if not _mesh_cache:
        _mesh_cache.append(
            plsc.VectorSubcoreMesh(core_axis_name="c", subcore_axis_name="s"))
    return _mesh_cache[0]

_PREC = lax.Precision.DEFAULT


def _leaky(x, s=0.01):
    return jnp.where(x >= 0, x, s * x)


def _mm(a, b):
    return lax.dot_general(a, b, (((1,), (0,)), ((), ())), precision=_PREC)


def _gdb_T(segs, layers):
    norms = [None] * len(segs)
    for (g, b, wt) in layers:
        off = 0
        y = None
        for j, s in enumerate(segs):
            cj = s.shape[0]
            if norms[j] is None:
                mu = jnp.mean(s, axis=1, keepdims=True)
                d = s - mu
                var = jnp.mean(d * d, axis=1, keepdims=True)
                norms[j] = d / jnp.sqrt(var + 1e-5)
            t = _leaky(g[off:off + cj] * norms[j] + b[off:off + cj])
            contrib = _mm(wt[:, off:off + cj], t)
            y = contrib if y is None else y + contrib
            off += cj
        segs = segs + [y]
        norms = norms + [None]
    return segs


def _gat_proj(segs, wt, a1t, a2t):
    off = 0
    ht = None
    for s in segs:
        cj = s.shape[0]
        contrib = _mm(wt[:, off:off + cj], s)
        ht = contrib if ht is None else ht + contrib
        off += cj
    p = _mm(a1t, ht)
    q = _mm(a2t, ht)
    pq = jnp.concatenate([p, q], axis=0)
    return ht, pq


def _tc1_body(x_t_ref, w1t, b1, w2t, b2, w3t, b3,
              g1, be1, wt1, g2, be2, wt2, g3, be3, wt3,
              wtg, a1t, a2t,
              ht_out, pq_out):
    xt = x_t_ref[...]
    h = _leaky(_mm(w1t[...], xt) + b1[...])
    h = _leaky(_mm(w2t[...], h) + b2[...])
    h = _leaky(_mm(w3t[...], h) + b3[...])
    segs = _gdb_T([h], [(g1[...], be1[...], wt1[...]),
                        (g2[...], be2[...], wt2[...]),
                        (g3[...], be3[...], wt3[...])])
    ht, pq = _gat_proj(segs, wtg[...], a1t[...], a2t[...])
    ht_out[...] = ht
    pq_out[...] = pq


def _tc_mid_body(raw_ref, den_ref, bias, g1, be1, wt1, g2, be2, wt2,
                 g3, be3, wt3, wtg, a1t, a2t, ht_out, pq_out):
    den = jnp.sum(den_ref[...], axis=0, keepdims=True)
    x0 = raw_ref[...] / (den + 1e-16) + bias[...]
    segs = _gdb_T([x0], [(g1[...], be1[...], wt1[...]),
                         (g2[...], be2[...], wt2[...]),
                         (g3[...], be3[...], wt3[...])])
    ht, pq = _gat_proj(segs, wtg[...], a1t[...], a2t[...])
    ht_out[...] = ht
    pq_out[...] = pq


def _tc3_body(raw_ref, den_ref, bias, g1, be1, wt1, g2, be2, wt2,
              g3, be3, wt3, o0, o1, o2, o3):
    den = jnp.sum(den_ref[...], axis=0, keepdims=True)
    x0 = raw_ref[...] / (den + 1e-16) + bias[...]
    segs = _gdb_T([x0], [(g1[...], be1[...], wt1[...]),
                         (g2[...], be2[...], wt2[...]),
                         (g3[...], be3[...], wt3[...])])
    o0[...] = segs[0]
    o1[...] = segs[1]
    o2[...] = segs[2]
    o3[...] = segs[3]


def _sca_body(pq_hbm, src_hbm, dst_hbm, ee_hbm, den_hbm,
              p_v, q_v, src_v, dst_v, ee_v, den_v):
    wid = lax.axis_index("c") * 16 + lax.axis_index("s")
    base = wid * EPT
    pltpu.sync_copy(pq_hbm.at[0], p_v)
    pltpu.sync_copy(pq_hbm.at[1], q_v)
    pltpu.sync_copy(src_hbm.at[pl.ds(base, EPT)], src_v)
    pltpu.sync_copy(dst_hbm.at[pl.ds(base, EPT)], dst_v)

    @pl.loop(0, NN, step=16)
    def _(i):
        den_v[pl.ds(i, 16)] = jnp.zeros((16,), jnp.float32)

    @pl.loop(0, EPT, step=16)
    def _(i):
        s16 = src_v[pl.ds(i, 16)]
        d16 = dst_v[pl.ds(i, 16)]
        e = plsc.load_gather(p_v, [d16]) + plsc.load_gather(q_v, [s16])
        e = jnp.where(e >= 0, e, SLOPE_GAT * e)
        ee = jnp.exp(e)
        ee_v[pl.ds(i, 16)] = ee
        plsc.addupdate_scatter(den_v, [d16], ee)

    pltpu.sync_copy(ee_v, ee_hbm.at[pl.ds(base, EPT)])
    pltpu.sync_copy(den_v, den_hbm.at[wid])


def _sc_phase_a(pq, src, dst):
    kern = pl.kernel(
        _sca_body,
        out_type=(jax.ShapeDtypeStruct((EE,), jnp.float32),
                  jax.ShapeDtypeStruct((NTILES, NN), jnp.float32)),
        mesh=_mesh(),
        scratch_types=[pltpu.VMEM((NN,), jnp.float32),
                       pltpu.VMEM((NN,), jnp.float32),
                       pltpu.VMEM((EPT,), jnp.int32),
                       pltpu.VMEM((EPT,), jnp.int32),
                       pltpu.VMEM((EPT,), jnp.float32),
                       pltpu.VMEM((NN,), jnp.float32)],
        compiler_params=_SC_CP,
    )
    return kern(pq, src, dst)


def _make_scb_body(nchan):
    def body(h_hbm, src_hbm, dst_hbm, ee_hbm, raw_hbm,
             hrow_v, orow_v, src_v, dst_v, ee_v):
        wid = lax.axis_index("c") * 16 + lax.axis_index("s")
        for k in range((nchan + NTILES - 1) // NTILES):
            c = wid + NTILES * k

            @pl.when(c < nchan)
            def _():
                pltpu.sync_copy(h_hbm.at[c], hrow_v)

                @pl.loop(0, NN, step=16)
                def _(i):
                    orow_v[pl.ds(i, 16)] = jnp.zeros((16,), jnp.float32)

                @pl.loop(0, EE, step=CHB)
                def _(eoff):
                    pltpu.sync_copy(src_hbm.at[pl.ds(eoff, CHB)], src_v)
                    pltpu.sync_copy(dst_hbm.at[pl.ds(eoff, CHB)], dst_v)
                    pltpu.sync_copy(ee_hbm.at[pl.ds(eoff, CHB)], ee_v)

                    @pl.loop(0, CHB, step=16)
                    def _(j):
                        s16 = src_v[pl.ds(j, 16)]
                        d16 = dst_v[pl.ds(j, 16)]
                        w16 = ee_v[pl.ds(j, 16)]
                        g = plsc.load_gather(hrow_v, [s16])
                        plsc.addupdate_scatter(orow_v, [d16], g * w16)

                pltpu.sync_copy(orow_v, raw_hbm.at[c])
    return body


def _sc_phase_b(ht, src, dst, ee):
    nchan = ht.shape[0]
    kern = pl.kernel(
        _make_scb_body(nchan),
        out_type=jax.ShapeDtypeStruct((nchan, NN), jnp.float32),
        mesh=_mesh(),
        scratch_types=[pltpu.VMEM((NN,), jnp.float32),
                       pltpu.VMEM((NN,), jnp.float32),
                       pltpu.VMEM((CHB,), jnp.int32),
                       pltpu.VMEM((CHB,), jnp.int32),
                       pltpu.VMEM((CHB,), jnp.float32)],
        compiler_params=_SC_CP,
    )
    return kern(ht, src, dst, ee)


def _col(v):
    return v.reshape(-1, 1)


def _block_args(block):
    out = []
    for (g, b, w) in block:
        out += [_col(g), _col(b), w.T]
    return out


def kernel(x, edge_index, params):
    src = edge_index[0]
    dst = edge_index[1]
    mlp = params['mlp']

    w_g1, a_g1, bias_g1 = params['trans1']
    w_g2, a_g2, bias_g2 = params['trans2']
    c1 = w_g1.shape[0]
    o1 = w_g1.shape[1]
    c2 = w_g2.shape[0]
    o2 = w_g2.shape[1]

    xt = x.T

    tc1 = pl.pallas_call(
        _tc1_body,
        out_shape=(jax.ShapeDtypeStruct((o1, NN), jnp.float32),
                   jax.ShapeDtypeStruct((2, NN), jnp.float32)),
        compiler_params=_TC_CP,
    )
    ht1, pq1 = tc1(
        xt, mlp['W1'].T, _col(mlp['b1']), mlp['W2'].T, _col(mlp['b2']),
        mlp['W3'].T, _col(mlp['b3']),
        *_block_args(params['block1']),
        w_g1.T, a_g1[:o1].T, a_g1[o1:].T,
    )

    ee1, den1 = _sc_phase_a(pq1, src, dst)
    raw1 = _sc_phase_b(ht1, src, dst, ee1)

    tc2 = pl.pallas_call(
        _tc_mid_body,
        out_shape=(jax.ShapeDtypeStruct((o2, NN), jnp.float32),
                   jax.ShapeDtypeStruct((2, NN), jnp.float32)),
        compiler_params=_TC_CP,
    )
    ht2, pq2 = tc2(
        raw1, den1, _col(bias_g1),
        *_block_args(params['block2']),
        w_g2.T, a_g2[:o2].T, a_g2[o2:].T,
    )

    ee2, den2 = _sc_phase_a(pq2, src, dst)
    raw2 = _sc_phase_b(ht2, src, dst, ee2)

    gr = params['block3'][0][2].shape[1]
    tc3 = pl.pallas_call(
        _tc3_body,
        out_shape=(jax.ShapeDtypeStruct((o2, NN), jnp.float32),
                   jax.ShapeDtypeStruct((gr, NN), jnp.float32),
                   jax.ShapeDtypeStruct((gr, NN), jnp.float32),
                   jax.ShapeDtypeStruct((gr, NN), jnp.float32)),
        compiler_params=_TC_CP,
    )
    s0, s1, s2, s3 = tc3(
        raw2, den2, _col(bias_g2),
        *_block_args(params['block3']),
    )

    return jnp.concatenate([s0.T, s1.T, s2.T, s3.T], axis=1)


# speedup vs baseline: 21.9008x; 2.8972x over previous
import dataclasses
import jax
import jax.numpy as jnp
from jax import lax
from jax.experimental import pallas as pl
from jax.experimental.pallas import tpu as pltpu
from jax.experimental.pallas import tpu_sc as plsc

NN = 10000
EE = 320000
SLOPE_GAT = 0.2
NTILES = 32
EPT = EE // NTILES
CHB = 4000

_SC_CP = pltpu.CompilerParams()
if "needs_layout_passes" in pltpu.CompilerParams.__dataclass_fields__:
    _SC_CP = dataclasses.replace(_SC_CP, needs_layout_passes=False)
_TC_CP = pltpu.CompilerParams(vmem_limit_bytes=100 * 1024 * 1024)

_mesh_cache = []


def _mesh():
    if not _mesh_cache:
        _mesh_cache.append(
            plsc.VectorSubcoreMesh(core_axis_name="c", subcore_axis_name="s"))
    return _mesh_cache[0]

_PREC = lax.Precision.DEFAULT


def _leaky(x, s=0.01):
    return jnp.where(x >= 0, x, s * x)


def _mm(a, b):
    return lax.dot_general(a, b, (((1,), (0,)), ((), ())), precision=_PREC)


def _gdb_T(segs, layers):
    norms = [None] * len(segs)
    for (g, b, wt) in layers:
        off = 0
        y = None
        for j, s in enumerate(segs):
            cj = s.shape[0]
            if norms[j] is None:
                mu = jnp.mean(s, axis=1, keepdims=True)
                d = s - mu
                var = jnp.mean(d * d, axis=1, keepdims=True)
                norms[j] = d / jnp.sqrt(var + 1e-5)
            t = _leaky(g[off:off + cj] * norms[j] + b[off:off + cj])
            contrib = _mm(wt[:, off:off + cj], t)
            y = contrib if y is None else y + contrib
            off += cj
        segs = segs + [y]
        norms = norms + [None]
    return segs


def _gat_proj(segs, wt, a1t, a2t):
    off = 0
    ht = None
    for s in segs:
        cj = s.shape[0]
        contrib = _mm(wt[:, off:off + cj], s)
        ht = contrib if ht is None else ht + contrib
        off += cj
    p = _mm(a1t, ht)
    q = _mm(a2t, ht)
    pq = jnp.concatenate([p, q], axis=0)
    return ht, pq


def _tc1_body(x_t_ref, w1t, b1, w2t, b2, w3t, b3,
              g1, be1, wt1, g2, be2, wt2, g3, be3, wt3,
              wtg, a1t, a2t,
              ht_out, pq_out):
    xt = x_t_ref[...]
    h = _leaky(_mm(w1t[...], xt) + b1[...])
    h = _leaky(_mm(w2t[...], h) + b2[...])
    h = _leaky(_mm(w3t[...], h) + b3[...])
    segs = _gdb_T([h], [(g1[...], be1[...], wt1[...]),
                        (g2[...], be2[...], wt2[...]),
                        (g3[...], be3[...], wt3[...])])
    ht, pq = _gat_proj(segs, wtg[...], a1t[...], a2t[...])
    ht_out[...] = ht
    pq_out[...] = pq


def _tc_mid_body(raw_ref, den_ref, bias, g1, be1, wt1, g2, be2, wt2,
                 g3, be3, wt3, wtg, a1t, a2t, ht_out, pq_out):
    den = jnp.sum(den_ref[...], axis=0, keepdims=True)
    raw = raw_ref[0] + raw_ref[1]
    x0 = raw / (den + 1e-16) + bias[...]
    segs = _gdb_T([x0], [(g1[...], be1[...], wt1[...]),
                         (g2[...], be2[...], wt2[...]),
                         (g3[...], be3[...], wt3[...])])
    ht, pq = _gat_proj(segs, wtg[...], a1t[...], a2t[...])
    ht_out[...] = ht
    pq_out[...] = pq


def _tc3_body(raw_ref, den_ref, bias, g1, be1, wt1, g2, be2, wt2,
              g3, be3, wt3, o0, o1, o2, o3):
    den = jnp.sum(den_ref[...], axis=0, keepdims=True)
    raw = raw_ref[0] + raw_ref[1]
    x0 = raw / (den + 1e-16) + bias[...]
    segs = _gdb_T([x0], [(g1[...], be1[...], wt1[...]),
                         (g2[...], be2[...], wt2[...]),
                         (g3[...], be3[...], wt3[...])])
    o0[...] = segs[0]
    o1[...] = segs[1]
    o2[...] = segs[2]
    o3[...] = segs[3]


def _sca_body(pq_hbm, src_hbm, dst_hbm, ee_hbm, den_hbm,
              p_v, q_v, src_v, dst_v, ee_v, den_v):
    wid = lax.axis_index("c") * 16 + lax.axis_index("s")
    base = wid * EPT
    pltpu.sync_copy(pq_hbm.at[0], p_v)
    pltpu.sync_copy(pq_hbm.at[1], q_v)
    pltpu.sync_copy(src_hbm.at[pl.ds(base, EPT)], src_v)
    pltpu.sync_copy(dst_hbm.at[pl.ds(base, EPT)], dst_v)

    @pl.loop(0, NN, step=16)
    def _(i):
        den_v[pl.ds(i, 16)] = jnp.zeros((16,), jnp.float32)

    @pl.loop(0, EPT, step=16)
    def _(i):
        s16 = src_v[pl.ds(i, 16)]
        d16 = dst_v[pl.ds(i, 16)]
        e = plsc.load_gather(p_v, [d16]) + plsc.load_gather(q_v, [s16])
        e = jnp.where(e >= 0, e, SLOPE_GAT * e)
        ee = jnp.exp(e)
        ee_v[pl.ds(i, 16)] = ee
        plsc.addupdate_scatter(den_v, [d16], ee)

    pltpu.sync_copy(ee_v, ee_hbm.at[pl.ds(base, EPT)])
    pltpu.sync_copy(den_v, den_hbm.at[wid])


def _sc_phase_a(pq, src, dst):
    kern = pl.kernel(
        _sca_body,
        out_type=(jax.ShapeDtypeStruct((EE,), jnp.float32),
                  jax.ShapeDtypeStruct((NTILES, NN), jnp.float32)),
        mesh=_mesh(),
        scratch_types=[pltpu.VMEM((NN,), jnp.float32),
                       pltpu.VMEM((NN,), jnp.float32),
                       pltpu.VMEM((EPT,), jnp.int32),
                       pltpu.VMEM((EPT,), jnp.int32),
                       pltpu.VMEM((EPT,), jnp.float32),
                       pltpu.VMEM((NN,), jnp.float32)],
        compiler_params=_SC_CP,
    )
    return kern(pq, src, dst)


NRANGE = 2
ERANGE = EE // NRANGE
NGRP = 16
CHB2 = 2000
_NCHUNK = ERANGE // CHB2


def _make_scb_body(nchan):
    nch_max = (nchan + NGRP - 1) // NGRP

    def body(h_hbm, src_hbm, dst_hbm, ee_hbm, raw_hbm, *scr):
        h_refs = scr[0:nch_max]
        o_refs = scr[nch_max:2 * nch_max]
        sbufs = scr[2 * nch_max:2 * nch_max + 2]
        dbufs = scr[2 * nch_max + 2:2 * nch_max + 4]
        ebufs = scr[2 * nch_max + 4:2 * nch_max + 6]
        sems = scr[2 * nch_max + 6:2 * nch_max + 8]

        r = lax.axis_index("c")
        g = lax.axis_index("s")
        ebase = r * ERANGE

        for j in range(nch_max):
            c_eff = jnp.minimum(g + NGRP * j, nchan - 1)
            pltpu.sync_copy(h_hbm.at[c_eff], h_refs[j])

        @pl.loop(0, NN, step=16)
        def _(i):
            for j in range(nch_max):
                o_refs[j][pl.ds(i, 16)] = jnp.zeros((16,), jnp.float32)

        def start(chunk, b):
            off = ebase + chunk * CHB2
            pltpu.async_copy(src_hbm.at[pl.ds(off, CHB2)], sbufs[b], sems[b])
            pltpu.async_copy(dst_hbm.at[pl.ds(off, CHB2)], dbufs[b], sems[b])
            pltpu.async_copy(ee_hbm.at[pl.ds(off, CHB2)], ebufs[b], sems[b])

        def wait(b):
            pltpu.make_async_copy(src_hbm.at[pl.ds(0, CHB2)], sbufs[b],
                                  sems[b]).wait()
            pltpu.make_async_copy(dst_hbm.at[pl.ds(0, CHB2)], dbufs[b],
                                  sems[b]).wait()
            pltpu.make_async_copy(ee_hbm.at[pl.ds(0, CHB2)], ebufs[b],
                                  sems[b]).wait()

        def process(b):
            @pl.loop(0, CHB2, step=16)
            def _(j):
                s16 = sbufs[b][pl.ds(j, 16)]
                d16 = dbufs[b][pl.ds(j, 16)]
                w16 = ebufs[b][pl.ds(j, 16)]
                for jch in range(nch_max):
                    gat = plsc.load_gather(h_refs[jch], [s16])
                    plsc.addupdate_scatter(o_refs[jch], [d16], gat * w16)

        start(0, 0)

        @pl.loop(0, _NCHUNK, step=2)
        def _(i):
            start(i + 1, 1)
            wait(0)
            process(0)

            @pl.when(i + 2 < _NCHUNK)
            def _():
                start(i + 2, 0)

            wait(1)
            process(1)

        for j in range(nch_max):
            c = g + NGRP * j

            @pl.when(c < nchan)
            def _():
                pltpu.sync_copy(o_refs[j], raw_hbm.at[r, c])

    return body


def _sc_phase_b(ht, src, dst, ee):
    nchan = ht.shape[0]
    nch_max = (nchan + NGRP - 1) // NGRP
    kern = pl.kernel(
        _make_scb_body(nchan),
        out_type=jax.ShapeDtypeStruct((NRANGE, nchan, NN), jnp.float32),
        mesh=_mesh(),
        scratch_types=([pltpu.VMEM((NN,), jnp.float32)] * (2 * nch_max)
                       + [pltpu.VMEM((CHB2,), jnp.int32)] * 4
                       + [pltpu.VMEM((CHB2,), jnp.float32)] * 2
                       + [pltpu.SemaphoreType.DMA] * 2),
        compiler_params=_SC_CP,
    )
    return kern(ht, src, dst, ee)


def _col(v):
    return v.reshape(-1, 1)


def _block_args(block):
    out = []
    for (g, b, w) in block:
        out += [_col(g), _col(b), w.T]
    return out


def kernel(x, edge_index, params):
    src = edge_index[0]
    dst = edge_index[1]
    mlp = params['mlp']

    w_g1, a_g1, bias_g1 = params['trans1']
    w_g2, a_g2, bias_g2 = params['trans2']
    c1 = w_g1.shape[0]
    o1 = w_g1.shape[1]
    c2 = w_g2.shape[0]
    o2 = w_g2.shape[1]

    xt = x.T

    tc1 = pl.pallas_call(
        _tc1_body,
        out_shape=(jax.ShapeDtypeStruct((o1, NN), jnp.float32),
                   jax.ShapeDtypeStruct((2, NN), jnp.float32)),
        compiler_params=_TC_CP,
    )
    ht1, pq1 = tc1(
        xt, mlp['W1'].T, _col(mlp['b1']), mlp['W2'].T, _col(mlp['b2']),
        mlp['W3'].T, _col(mlp['b3']),
        *_block_args(params['block1']),
        w_g1.T, a_g1[:o1].T, a_g1[o1:].T,
    )

    ee1, den1 = _sc_phase_a(pq1, src, dst)
    raw1 = _sc_phase_b(ht1, src, dst, ee1)

    tc2 = pl.pallas_call(
        _tc_mid_body,
        out_shape=(jax.ShapeDtypeStruct((o2, NN), jnp.float32),
                   jax.ShapeDtypeStruct((2, NN), jnp.float32)),
        compiler_params=_TC_CP,
    )
    ht2, pq2 = tc2(
        raw1, den1, _col(bias_g1),
        *_block_args(params['block2']),
        w_g2.T, a_g2[:o2].T, a_g2[o2:].T,
    )

    ee2, den2 = _sc_phase_a(pq2, src, dst)
    raw2 = _sc_phase_b(ht2, src, dst, ee2)

    gr = params['block3'][0][2].shape[1]
    tc3 = pl.pallas_call(
        _tc3_body,
        out_shape=(jax.ShapeDtypeStruct((o2, NN), jnp.float32),
                   jax.ShapeDtypeStruct((gr, NN), jnp.float32),
                   jax.ShapeDtypeStruct((gr, NN), jnp.float32),
                   jax.ShapeDtypeStruct((gr, NN), jnp.float32)),
        compiler_params=_TC_CP,
    )
    s0, s1, s2, s3 = tc3(
        raw2, den2, _col(bias_g2),
        *_block_args(params['block3']),
    )

    return jnp.concatenate([s0.T, s1.T, s2.T, s3.T], axis=1)


# speedup vs baseline: 44.7338x; 2.0426x over previous
import dataclasses
import jax
import jax.numpy as jnp
from jax import lax
from jax.experimental import pallas as pl
from jax.experimental.pallas import tpu as pltpu
from jax.experimental.pallas import tpu_sc as plsc

NN = 10000
EE = 320000
SLOPE_GAT = 0.2
NTILES = 32
EPT = EE // NTILES
CHB = 4000

_SC_CP = pltpu.CompilerParams()
if "needs_layout_passes" in pltpu.CompilerParams.__dataclass_fields__:
    _SC_CP = dataclasses.replace(_SC_CP, needs_layout_passes=False)
_TC_CP = pltpu.CompilerParams(vmem_limit_bytes=100 * 1024 * 1024)

_mesh_cache = []


def _mesh():
    if not _mesh_cache:
        _mesh_cache.append(
            plsc.VectorSubcoreMesh(core_axis_name="c", subcore_axis_name="s"))
    return _mesh_cache[0]

_PREC = lax.Precision.DEFAULT


def _leaky(x, s=0.01):
    return jnp.where(x >= 0, x, s * x)


def _mm(a, b):
    return lax.dot_general(a, b, (((1,), (0,)), ((), ())), precision=_PREC)


def _gdb_T(segs, layers):
    norms = [None] * len(segs)
    for (g, b, wt) in layers:
        off = 0
        y = None
        for j, s in enumerate(segs):
            cj = s.shape[0]
            if norms[j] is None:
                mu = jnp.mean(s, axis=1, keepdims=True)
                d = s - mu
                var = jnp.mean(d * d, axis=1, keepdims=True)
                norms[j] = d / jnp.sqrt(var + 1e-5)
            t = _leaky(g[off:off + cj] * norms[j] + b[off:off + cj])
            contrib = _mm(wt[:, off:off + cj], t)
            y = contrib if y is None else y + contrib
            off += cj
        segs = segs + [y]
        norms = norms + [None]
    return segs


def _gat_proj(segs, wt, a1t, a2t):
    off = 0
    ht = None
    for s in segs:
        cj = s.shape[0]
        contrib = _mm(wt[:, off:off + cj], s)
        ht = contrib if ht is None else ht + contrib
        off += cj
    p = _mm(a1t, ht)
    q = _mm(a2t, ht)
    pq = jnp.concatenate([p, q], axis=0)
    return ht, pq


def _tc1_body(x_t_ref, w1t, b1, w2t, b2, w3t, b3,
              g1, be1, wt1, g2, be2, wt2, g3, be3, wt3,
              wtg, a1t, a2t,
              ht_out, pq_out):
    xt = x_t_ref[...]
    h = _leaky(_mm(w1t[...], xt) + b1[...])
    h = _leaky(_mm(w2t[...], h) + b2[...])
    h = _leaky(_mm(w3t[...], h) + b3[...])
    segs = _gdb_T([h], [(g1[...], be1[...], wt1[...]),
                        (g2[...], be2[...], wt2[...]),
                        (g3[...], be3[...], wt3[...])])
    ht, pq = _gat_proj(segs, wtg[...], a1t[...], a2t[...])
    ht_out[...] = ht
    pq_out[...] = pq


def _tc_mid_body(raw_ref, den_ref, bias, g1, be1, wt1, g2, be2, wt2,
                 g3, be3, wt3, wtg, a1t, a2t, ht_out, pq_out):
    den = jnp.sum(den_ref[...], axis=0, keepdims=True)
    raw = raw_ref[0] + raw_ref[1]
    x0 = raw / (den + 1e-16) + bias[...]
    segs = _gdb_T([x0], [(g1[...], be1[...], wt1[...]),
                         (g2[...], be2[...], wt2[...]),
                         (g3[...], be3[...], wt3[...])])
    ht, pq = _gat_proj(segs, wtg[...], a1t[...], a2t[...])
    ht_out[...] = ht
    pq_out[...] = pq


def _tc3_body(raw_ref, den_ref, bias, g1, be1, wt1, g2, be2, wt2,
              g3, be3, wt3, o0, o1, o2, o3):
    den = jnp.sum(den_ref[...], axis=0, keepdims=True)
    raw = raw_ref[0] + raw_ref[1]
    x0 = raw / (den + 1e-16) + bias[...]
    segs = _gdb_T([x0], [(g1[...], be1[...], wt1[...]),
                         (g2[...], be2[...], wt2[...]),
                         (g3[...], be3[...], wt3[...])])
    o0[...] = segs[0]
    o1[...] = segs[1]
    o2[...] = segs[2]
    o3[...] = segs[3]


def _sca_body(pq_hbm, src_hbm, dst_hbm, ee_hbm, den_hbm,
              p_v, q_v, src_v, dst_v, ee_v, den_v):
    wid = lax.axis_index("c") * 16 + lax.axis_index("s")
    base = wid * EPT
    pltpu.sync_copy(pq_hbm.at[0], p_v)
    pltpu.sync_copy(pq_hbm.at[1], q_v)
    pltpu.sync_copy(src_hbm.at[pl.ds(base, EPT)], src_v)
    pltpu.sync_copy(dst_hbm.at[pl.ds(base, EPT)], dst_v)

    @pl.loop(0, NN, step=16)
    def _(i):
        den_v[pl.ds(i, 16)] = jnp.zeros((16,), jnp.float32)

    @plsc.parallel_loop(0, EPT, step=16, unroll=4)
    def _(i):
        s16 = src_v[pl.ds(i, 16)]
        d16 = dst_v[pl.ds(i, 16)]
        e = plsc.load_gather(p_v, [d16]) + plsc.load_gather(q_v, [s16])
        e = jnp.where(e >= 0, e, SLOPE_GAT * e)
        ee = jnp.exp(e)
        ee_v[pl.ds(i, 16)] = ee
        plsc.addupdate_scatter(den_v, [d16], ee)

    pltpu.sync_copy(ee_v, ee_hbm.at[pl.ds(base, EPT)])
    pltpu.sync_copy(den_v, den_hbm.at[wid])


def _sc_phase_a(pq, src, dst):
    kern = pl.kernel(
        _sca_body,
        out_type=(jax.ShapeDtypeStruct((EE,), jnp.float32),
                  jax.ShapeDtypeStruct((NTILES, NN), jnp.float32)),
        mesh=_mesh(),
        scratch_types=[pltpu.VMEM((NN,), jnp.float32),
                       pltpu.VMEM((NN,), jnp.float32),
                       pltpu.VMEM((EPT,), jnp.int32),
                       pltpu.VMEM((EPT,), jnp.int32),
                       pltpu.VMEM((EPT,), jnp.float32),
                       pltpu.VMEM((NN,), jnp.float32)],
        compiler_params=_SC_CP,
    )
    return kern(pq, src, dst)


NRANGE = 2
ERANGE = EE // NRANGE
NGRP = 16
CHB2 = 2000
_NCHUNK = ERANGE // CHB2


def _make_scb_body(nchan):
    nch_max = (nchan + NGRP - 1) // NGRP

    def body(h_hbm, src_hbm, dst_hbm, ee_hbm, raw_hbm, *scr):
        h_refs = scr[0:nch_max]
        o_refs = scr[nch_max:2 * nch_max]
        sbufs = scr[2 * nch_max:2 * nch_max + 2]
        dbufs = scr[2 * nch_max + 2:2 * nch_max + 4]
        ebufs = scr[2 * nch_max + 4:2 * nch_max + 6]
        sems = scr[2 * nch_max + 6:2 * nch_max + 8]

        r = lax.axis_index("c")
        g = lax.axis_index("s")
        ebase = r * ERANGE

        for j in range(nch_max):
            c_eff = jnp.minimum(g + NGRP * j, nchan - 1)
            pltpu.sync_copy(h_hbm.at[c_eff], h_refs[j])

        @pl.loop(0, NN, step=16)
        def _(i):
            for j in range(nch_max):
                o_refs[j][pl.ds(i, 16)] = jnp.zeros((16,), jnp.float32)

        def start(chunk, b):
            off = ebase + chunk * CHB2
            pltpu.async_copy(src_hbm.at[pl.ds(off, CHB2)], sbufs[b], sems[b])
            pltpu.async_copy(dst_hbm.at[pl.ds(off, CHB2)], dbufs[b], sems[b])
            pltpu.async_copy(ee_hbm.at[pl.ds(off, CHB2)], ebufs[b], sems[b])

        def wait(b):
            pltpu.make_async_copy(src_hbm.at[pl.ds(0, CHB2)], sbufs[b],
                                  sems[b]).wait()
            pltpu.make_async_copy(dst_hbm.at[pl.ds(0, CHB2)], dbufs[b],
                                  sems[b]).wait()
            pltpu.make_async_copy(ee_hbm.at[pl.ds(0, CHB2)], ebufs[b],
                                  sems[b]).wait()

        def process(b):
            @plsc.parallel_loop(0, CHB2, step=16, unroll=4)
            def _(j):
                s16 = sbufs[b][pl.ds(j, 16)]
                d16 = dbufs[b][pl.ds(j, 16)]
                w16 = ebufs[b][pl.ds(j, 16)]
                for jch in range(nch_max):
                    gat = plsc.load_gather(h_refs[jch], [s16])
                    plsc.addupdate_scatter(o_refs[jch], [d16], gat * w16)

        start(0, 0)

        @pl.loop(0, _NCHUNK, step=2)
        def _(i):
            start(i + 1, 1)
            wait(0)
            process(0)

            @pl.when(i + 2 < _NCHUNK)
            def _():
                start(i + 2, 0)

            wait(1)
            process(1)

        for j in range(nch_max):
            c = g + NGRP * j

            @pl.when(c < nchan)
            def _():
                pltpu.sync_copy(o_refs[j], raw_hbm.at[r, c])

    return body


def _sc_phase_b(ht, src, dst, ee):
    nchan = ht.shape[0]
    nch_max = (nchan + NGRP - 1) // NGRP
    kern = pl.kernel(
        _make_scb_body(nchan),
        out_type=jax.ShapeDtypeStruct((NRANGE, nchan, NN), jnp.float32),
        mesh=_mesh(),
        scratch_types=([pltpu.VMEM((NN,), jnp.float32)] * (2 * nch_max)
                       + [pltpu.VMEM((CHB2,), jnp.int32)] * 4
                       + [pltpu.VMEM((CHB2,), jnp.float32)] * 2
                       + [pltpu.SemaphoreType.DMA] * 2),
        compiler_params=_SC_CP,
    )
    return kern(ht, src, dst, ee)


def _col(v):
    return v.reshape(-1, 1)


def _block_args(block):
    out = []
    for (g, b, w) in block:
        out += [_col(g), _col(b), w.T]
    return out


def kernel(x, edge_index, params):
    src = edge_index[0]
    dst = edge_index[1]
    mlp = params['mlp']

    w_g1, a_g1, bias_g1 = params['trans1']
    w_g2, a_g2, bias_g2 = params['trans2']
    c1 = w_g1.shape[0]
    o1 = w_g1.shape[1]
    c2 = w_g2.shape[0]
    o2 = w_g2.shape[1]

    xt = x.T

    tc1 = pl.pallas_call(
        _tc1_body,
        out_shape=(jax.ShapeDtypeStruct((o1, NN), jnp.float32),
                   jax.ShapeDtypeStruct((2, NN), jnp.float32)),
        compiler_params=_TC_CP,
    )
    ht1, pq1 = tc1(
        xt, mlp['W1'].T, _col(mlp['b1']), mlp['W2'].T, _col(mlp['b2']),
        mlp['W3'].T, _col(mlp['b3']),
        *_block_args(params['block1']),
        w_g1.T, a_g1[:o1].T, a_g1[o1:].T,
    )

    ee1, den1 = _sc_phase_a(pq1, src, dst)
    raw1 = _sc_phase_b(ht1, src, dst, ee1)

    tc2 = pl.pallas_call(
        _tc_mid_body,
        out_shape=(jax.ShapeDtypeStruct((o2, NN), jnp.float32),
                   jax.ShapeDtypeStruct((2, NN), jnp.float32)),
        compiler_params=_TC_CP,
    )
    ht2, pq2 = tc2(
        raw1, den1, _col(bias_g1),
        *_block_args(params['block2']),
        w_g2.T, a_g2[:o2].T, a_g2[o2:].T,
    )

    ee2, den2 = _sc_phase_a(pq2, src, dst)
    raw2 = _sc_phase_b(ht2, src, dst, ee2)

    gr = params['block3'][0][2].shape[1]
    tc3 = pl.pallas_call(
        _tc3_body,
        out_shape=(jax.ShapeDtypeStruct((o2, NN), jnp.float32),
                   jax.ShapeDtypeStruct((gr, NN), jnp.float32),
                   jax.ShapeDtypeStruct((gr, NN), jnp.float32),
                   jax.ShapeDtypeStruct((gr, NN), jnp.float32)),
        compiler_params=_TC_CP,
    )
    s0, s1, s2, s3 = tc3(
        raw2, den2, _col(bias_g2),
        *_block_args(params['block3']),
    )

    return jnp.concatenate([s0.T, s1.T, s2.T, s3.T], axis=1)


# speedup vs baseline: 47.9693x; 1.0723x over previous
import dataclasses
import jax
import jax.numpy as jnp
from jax import lax
from jax.experimental import pallas as pl
from jax.experimental.pallas import tpu as pltpu
from jax.experimental.pallas import tpu_sc as plsc

NN = 10000
EE = 320000
SLOPE_GAT = 0.2
NTILES = 32
EPT = EE // NTILES
CHB = 4000

_SC_CP = pltpu.CompilerParams()
if "needs_layout_passes" in pltpu.CompilerParams.__dataclass_fields__:
    _SC_CP = dataclasses.replace(_SC_CP, needs_layout_passes=False)
_TC_CP = pltpu.CompilerParams(vmem_limit_bytes=100 * 1024 * 1024)

_mesh_cache = []


def _mesh():
    if not _mesh_cache:
        _mesh_cache.append(
            plsc.VectorSubcoreMesh(core_axis_name="c", subcore_axis_name="s"))
    return _mesh_cache[0]

_PREC = lax.Precision.DEFAULT


def _leaky(x, s=0.01):
    return jnp.where(x >= 0, x, s * x)


def _mm(a, b):
    return lax.dot_general(a, b, (((1,), (0,)), ((), ())), precision=_PREC)


def _gdb_T(segs, layers):
    norms = [None] * len(segs)
    for (g, b, wt) in layers:
        off = 0
        y = None
        for j, s in enumerate(segs):
            cj = s.shape[0]
            if norms[j] is None:
                mu = jnp.mean(s, axis=1, keepdims=True)
                d = s - mu
                var = jnp.mean(d * d, axis=1, keepdims=True)
                norms[j] = d / jnp.sqrt(var + 1e-5)
            t = _leaky(g[off:off + cj] * norms[j] + b[off:off + cj])
            contrib = _mm(wt[:, off:off + cj], t)
            y = contrib if y is None else y + contrib
            off += cj
        segs = segs + [y]
        norms = norms + [None]
    return segs


def _gat_proj(segs, wt, a1t, a2t):
    off = 0
    ht = None
    for s in segs:
        cj = s.shape[0]
        contrib = _mm(wt[:, off:off + cj], s)
        ht = contrib if ht is None else ht + contrib
        off += cj
    p = _mm(a1t, ht)
    q = _mm(a2t, ht)
    pq = jnp.concatenate([p, q], axis=0)
    return ht, pq


def _tc1_body(x_t_ref, w1t, b1, w2t, b2, w3t, b3,
              g1, be1, wt1, g2, be2, wt2, g3, be3, wt3,
              wtg, a1t, a2t,
              ht_out, pq_out):
    xt = x_t_ref[...]
    h = _leaky(_mm(w1t[...], xt) + b1[...])
    h = _leaky(_mm(w2t[...], h) + b2[...])
    h = _leaky(_mm(w3t[...], h) + b3[...])
    segs = _gdb_T([h], [(g1[...], be1[...], wt1[...]),
                        (g2[...], be2[...], wt2[...]),
                        (g3[...], be3[...], wt3[...])])
    ht, pq = _gat_proj(segs, wtg[...], a1t[...], a2t[...])
    ht_out[...] = ht
    pq_out[...] = pq


def _tc_mid_body(raw_ref, den_ref, bias, g1, be1, wt1, g2, be2, wt2,
                 g3, be3, wt3, wtg, a1t, a2t, ht_out, pq_out):
    den = jnp.sum(den_ref[...], axis=0, keepdims=True)
    raw = jnp.sum(raw_ref[...], axis=0)
    x0 = raw / (den + 1e-16) + bias[...]
    segs = _gdb_T([x0], [(g1[...], be1[...], wt1[...]),
                         (g2[...], be2[...], wt2[...]),
                         (g3[...], be3[...], wt3[...])])
    ht, pq = _gat_proj(segs, wtg[...], a1t[...], a2t[...])
    ht_out[...] = ht
    pq_out[...] = pq


def _tc3_body(raw_ref, den_ref, bias, g1, be1, wt1, g2, be2, wt2,
              g3, be3, wt3, o0, o1, o2, o3):
    den = jnp.sum(den_ref[...], axis=0, keepdims=True)
    raw = jnp.sum(raw_ref[...], axis=0)
    x0 = raw / (den + 1e-16) + bias[...]
    segs = _gdb_T([x0], [(g1[...], be1[...], wt1[...]),
                         (g2[...], be2[...], wt2[...]),
                         (g3[...], be3[...], wt3[...])])
    o0[...] = segs[0]
    o1[...] = segs[1]
    o2[...] = segs[2]
    o3[...] = segs[3]


def _sca_body(pq_hbm, src_hbm, dst_hbm, ee_hbm, den_hbm,
              p_v, q_v, src_v, dst_v, ee_v, den_v):
    wid = lax.axis_index("c") * 16 + lax.axis_index("s")
    base = wid * EPT
    pltpu.sync_copy(pq_hbm.at[0], p_v)
    pltpu.sync_copy(pq_hbm.at[1], q_v)
    pltpu.sync_copy(src_hbm.at[pl.ds(base, EPT)], src_v)
    pltpu.sync_copy(dst_hbm.at[pl.ds(base, EPT)], dst_v)

    @pl.loop(0, NN, step=16)
    def _(i):
        den_v[pl.ds(i, 16)] = jnp.zeros((16,), jnp.float32)

    @plsc.parallel_loop(0, EPT, step=16, unroll=4)
    def _(i):
        s16 = src_v[pl.ds(i, 16)]
        d16 = dst_v[pl.ds(i, 16)]
        e = plsc.load_gather(p_v, [d16]) + plsc.load_gather(q_v, [s16])
        e = jnp.where(e >= 0, e, SLOPE_GAT * e)
        ee = jnp.exp(e)
        ee_v[pl.ds(i, 16)] = ee
        plsc.addupdate_scatter(den_v, [d16], ee)

    pltpu.sync_copy(ee_v, ee_hbm.at[pl.ds(base, EPT)])
    pltpu.sync_copy(den_v, den_hbm.at[wid])


def _sc_phase_a(pq, src, dst):
    kern = pl.kernel(
        _sca_body,
        out_type=(jax.ShapeDtypeStruct((EE,), jnp.float32),
                  jax.ShapeDtypeStruct((NTILES, NN), jnp.float32)),
        mesh=_mesh(),
        scratch_types=[pltpu.VMEM((NN,), jnp.float32),
                       pltpu.VMEM((NN,), jnp.float32),
                       pltpu.VMEM((EPT,), jnp.int32),
                       pltpu.VMEM((EPT,), jnp.int32),
                       pltpu.VMEM((EPT,), jnp.float32),
                       pltpu.VMEM((NN,), jnp.float32)],
        compiler_params=_SC_CP,
    )
    return kern(pq, src, dst)


def _scb_cfg(nchan):
    if nchan <= 40:
        return 4, 8, 1600
    return 4, 8, 800


def _make_scb_body(nchan):
    nrange, ngrp, chb = _scb_cfg(nchan)
    erange = EE // nrange
    nchunk = erange // chb
    nch_max = (nchan + ngrp - 1) // ngrp

    def body(h_hbm, src_hbm, dst_hbm, ee_hbm, raw_hbm, *scr):
        h_refs = scr[0:nch_max]
        o_refs = scr[nch_max:2 * nch_max]
        sbufs = scr[2 * nch_max:2 * nch_max + 2]
        dbufs = scr[2 * nch_max + 2:2 * nch_max + 4]
        ebufs = scr[2 * nch_max + 4:2 * nch_max + 6]
        sems = scr[2 * nch_max + 6:2 * nch_max + 8]

        core = lax.axis_index("c")
        sub = lax.axis_index("s")
        g = jnp.bitwise_and(sub, ngrp - 1)
        r = core * (nrange // 2) + lax.shift_right_logical(
            sub, ngrp.bit_length() - 1)
        ebase = r * erange

        for j in range(nch_max):
            c_eff = jnp.minimum(g + ngrp * j, nchan - 1)
            pltpu.sync_copy(h_hbm.at[c_eff], h_refs[j])

        @pl.loop(0, NN, step=16)
        def _(i):
            for j in range(nch_max):
                o_refs[j][pl.ds(i, 16)] = jnp.zeros((16,), jnp.float32)

        def start(chunk, b):
            off = ebase + chunk * chb
            pltpu.async_copy(src_hbm.at[pl.ds(off, chb)], sbufs[b], sems[b])
            pltpu.async_copy(dst_hbm.at[pl.ds(off, chb)], dbufs[b], sems[b])
            pltpu.async_copy(ee_hbm.at[pl.ds(off, chb)], ebufs[b], sems[b])

        def wait(b):
            pltpu.make_async_copy(src_hbm.at[pl.ds(0, chb)], sbufs[b],
                                  sems[b]).wait()
            pltpu.make_async_copy(dst_hbm.at[pl.ds(0, chb)], dbufs[b],
                                  sems[b]).wait()
            pltpu.make_async_copy(ee_hbm.at[pl.ds(0, chb)], ebufs[b],
                                  sems[b]).wait()

        def process(b):
            @plsc.parallel_loop(0, chb, step=16, unroll=4)
            def _(j):
                s16 = sbufs[b][pl.ds(j, 16)]
                d16 = dbufs[b][pl.ds(j, 16)]
                w16 = ebufs[b][pl.ds(j, 16)]
                for jch in range(nch_max):
                    gat = plsc.load_gather(h_refs[jch], [s16])
                    plsc.addupdate_scatter(o_refs[jch], [d16], gat * w16)

        start(0, 0)

        @pl.loop(0, nchunk, step=2)
        def _(i):
            start(i + 1, 1)
            wait(0)
            process(0)

            @pl.when(i + 2 < nchunk)
            def _():
                start(i + 2, 0)

            wait(1)
            process(1)

        for j in range(nch_max):
            c = g + ngrp * j

            @pl.when(c < nchan)
            def _():
                pltpu.sync_copy(o_refs[j], raw_hbm.at[r, c])

    return body


def _sc_phase_b(ht, src, dst, ee):
    nchan = ht.shape[0]
    nrange, ngrp, chb = _scb_cfg(nchan)
    nch_max = (nchan + ngrp - 1) // ngrp
    kern = pl.kernel(
        _make_scb_body(nchan),
        out_type=jax.ShapeDtypeStruct((nrange, nchan, NN), jnp.float32),
        mesh=_mesh(),
        scratch_types=([pltpu.VMEM((NN,), jnp.float32)] * (2 * nch_max)
                       + [pltpu.VMEM((chb,), jnp.int32)] * 4
                       + [pltpu.VMEM((chb,), jnp.float32)] * 2
                       + [pltpu.SemaphoreType.DMA] * 2),
        compiler_params=_SC_CP,
    )
    return kern(ht, src, dst, ee)


def _col(v):
    return v.reshape(-1, 1)


def _block_args(block):
    out = []
    for (g, b, w) in block:
        out += [_col(g), _col(b), w.T]
    return out


def kernel(x, edge_index, params):
    src = edge_index[0]
    dst = edge_index[1]
    mlp = params['mlp']

    w_g1, a_g1, bias_g1 = params['trans1']
    w_g2, a_g2, bias_g2 = params['trans2']
    c1 = w_g1.shape[0]
    o1 = w_g1.shape[1]
    c2 = w_g2.shape[0]
    o2 = w_g2.shape[1]

    xt = x.T

    tc1 = pl.pallas_call(
        _tc1_body,
        out_shape=(jax.ShapeDtypeStruct((o1, NN), jnp.float32),
                   jax.ShapeDtypeStruct((2, NN), jnp.float32)),
        compiler_params=_TC_CP,
    )
    ht1, pq1 = tc1(
        xt, mlp['W1'].T, _col(mlp['b1']), mlp['W2'].T, _col(mlp['b2']),
        mlp['W3'].T, _col(mlp['b3']),
        *_block_args(params['block1']),
        w_g1.T, a_g1[:o1].T, a_g1[o1:].T,
    )

    ee1, den1 = _sc_phase_a(pq1, src, dst)
    raw1 = _sc_phase_b(ht1, src, dst, ee1)

    tc2 = pl.pallas_call(
        _tc_mid_body,
        out_shape=(jax.ShapeDtypeStruct((o2, NN), jnp.float32),
                   jax.ShapeDtypeStruct((2, NN), jnp.float32)),
        compiler_params=_TC_CP,
    )
    ht2, pq2 = tc2(
        raw1, den1, _col(bias_g1),
        *_block_args(params['block2']),
        w_g2.T, a_g2[:o2].T, a_g2[o2:].T,
    )

    ee2, den2 = _sc_phase_a(pq2, src, dst)
    raw2 = _sc_phase_b(ht2, src, dst, ee2)

    gr = params['block3'][0][2].shape[1]
    tc3 = pl.pallas_call(
        _tc3_body,
        out_shape=(jax.ShapeDtypeStruct((o2, NN), jnp.float32),
                   jax.ShapeDtypeStruct((gr, NN), jnp.float32),
                   jax.ShapeDtypeStruct((gr, NN), jnp.float32),
                   jax.ShapeDtypeStruct((gr, NN), jnp.float32)),
        compiler_params=_TC_CP,
    )
    s0, s1, s2, s3 = tc3(
        raw2, den2, _col(bias_g2),
        *_block_args(params['block3']),
    )

    return jnp.concatenate([s0.T, s1.T, s2.T, s3.T], axis=1)


# speedup vs baseline: 49.0466x; 1.0225x over previous
import dataclasses
import jax
import jax.numpy as jnp
from jax import lax
from jax.experimental import pallas as pl
from jax.experimental.pallas import tpu as pltpu
from jax.experimental.pallas import tpu_sc as plsc

NN = 10000
EE = 320000
SLOPE_GAT = 0.2
NTILES = 32
EPT = EE // NTILES
CHB = 4000

_SC_CP = pltpu.CompilerParams()
if "needs_layout_passes" in pltpu.CompilerParams.__dataclass_fields__:
    _SC_CP = dataclasses.replace(_SC_CP, needs_layout_passes=False)
_TC_CP = pltpu.CompilerParams(vmem_limit_bytes=100 * 1024 * 1024)

_mesh_cache = []


def _mesh():
    if not _mesh_cache:
        _mesh_cache.append(
            plsc.VectorSubcoreMesh(core_axis_name="c", subcore_axis_name="s"))
    return _mesh_cache[0]

_PREC = lax.Precision.DEFAULT


def _leaky(x, s=0.01):
    return jnp.where(x >= 0, x, s * x)


def _mm(a, b):
    return lax.dot_general(a, b, (((1,), (0,)), ((), ())), precision=_PREC)


def _gdb_T(segs, layers):
    norms = [None] * len(segs)
    for (g, b, wt) in layers:
        off = 0
        y = None
        for j, s in enumerate(segs):
            cj = s.shape[0]
            if norms[j] is None:
                mu = jnp.mean(s, axis=1, keepdims=True)
                d = s - mu
                var = jnp.mean(d * d, axis=1, keepdims=True)
                norms[j] = d / jnp.sqrt(var + 1e-5)
            t = _leaky(g[off:off + cj] * norms[j] + b[off:off + cj])
            contrib = _mm(wt[:, off:off + cj], t)
            y = contrib if y is None else y + contrib
            off += cj
        segs = segs + [y]
        norms = norms + [None]
    return segs


def _gat_proj(segs, wt, a1t, a2t):
    off = 0
    ht = None
    for s in segs:
        cj = s.shape[0]
        contrib = _mm(wt[:, off:off + cj], s)
        ht = contrib if ht is None else ht + contrib
        off += cj
    p = _mm(a1t, ht)
    q = _mm(a2t, ht)
    pq = jnp.concatenate([p, q], axis=0)
    return ht, pq


def _tc1_body(x_t_ref, w1t, b1, w2t, b2, w3t, b3,
              g1, be1, wt1, g2, be2, wt2, g3, be3, wt3,
              wtg, a1t, a2t,
              ht_out, pq_out):
    xt = x_t_ref[...]
    h = _leaky(_mm(w1t[...], xt) + b1[...])
    h = _leaky(_mm(w2t[...], h) + b2[...])
    h = _leaky(_mm(w3t[...], h) + b3[...])
    segs = _gdb_T([h], [(g1[...], be1[...], wt1[...]),
                        (g2[...], be2[...], wt2[...]),
                        (g3[...], be3[...], wt3[...])])
    ht, pq = _gat_proj(segs, wtg[...], a1t[...], a2t[...])
    ht_out[...] = ht
    pq_out[...] = pq


def _tc_mid_body(raw_ref, den_ref, bias, g1, be1, wt1, g2, be2, wt2,
                 g3, be3, wt3, wtg, a1t, a2t, ht_out, pq_out):
    den = jnp.sum(den_ref[...], axis=0, keepdims=True)
    raw = jnp.sum(raw_ref[...], axis=0)
    x0 = raw / (den + 1e-16) + bias[...]
    segs = _gdb_T([x0], [(g1[...], be1[...], wt1[...]),
                         (g2[...], be2[...], wt2[...]),
                         (g3[...], be3[...], wt3[...])])
    ht, pq = _gat_proj(segs, wtg[...], a1t[...], a2t[...])
    ht_out[...] = ht
    pq_out[...] = pq


def _tc3_body(raw_ref, den_ref, bias, g1, be1, wt1, g2, be2, wt2,
              g3, be3, wt3, o0, o1, o2, o3):
    den = jnp.sum(den_ref[...], axis=0, keepdims=True)
    raw = jnp.sum(raw_ref[...], axis=0)
    x0 = raw / (den + 1e-16) + bias[...]
    segs = _gdb_T([x0], [(g1[...], be1[...], wt1[...]),
                         (g2[...], be2[...], wt2[...]),
                         (g3[...], be3[...], wt3[...])])
    o0[...] = segs[0]
    o1[...] = segs[1]
    o2[...] = segs[2]
    o3[...] = segs[3]


def _sca_body(pq_hbm, src_hbm, dst_hbm, ee_hbm, den_hbm,
              p_v, q_v, src_v, dst_v, ee_v, den_v, sem):
    wid = lax.axis_index("c") * 16 + lax.axis_index("s")
    base = wid * EPT
    pltpu.async_copy(pq_hbm.at[0], p_v, sem)
    pltpu.async_copy(pq_hbm.at[1], q_v, sem)
    pltpu.async_copy(src_hbm.at[pl.ds(base, EPT)], src_v, sem)
    pltpu.async_copy(dst_hbm.at[pl.ds(base, EPT)], dst_v, sem)
    pltpu.make_async_copy(pq_hbm.at[0], p_v, sem).wait()
    pltpu.make_async_copy(pq_hbm.at[1], q_v, sem).wait()
    pltpu.make_async_copy(src_hbm.at[pl.ds(0, EPT)], src_v, sem).wait()
    pltpu.make_async_copy(dst_hbm.at[pl.ds(0, EPT)], dst_v, sem).wait()

    @pl.loop(0, NN, step=16)
    def _(i):
        den_v[pl.ds(i, 16)] = jnp.zeros((16,), jnp.float32)

    @plsc.parallel_loop(0, EPT, step=16, unroll=4)
    def _(i):
        s16 = src_v[pl.ds(i, 16)]
        d16 = dst_v[pl.ds(i, 16)]
        e = plsc.load_gather(p_v, [d16]) + plsc.load_gather(q_v, [s16])
        e = jnp.where(e >= 0, e, SLOPE_GAT * e)
        ee = jnp.exp(e)
        ee_v[pl.ds(i, 16)] = ee
        plsc.addupdate_scatter(den_v, [d16], ee)

    pltpu.sync_copy(ee_v, ee_hbm.at[pl.ds(base, EPT)])
    pltpu.sync_copy(den_v, den_hbm.at[wid])


def _sc_phase_a(pq, src, dst):
    kern = pl.kernel(
        _sca_body,
        out_type=(jax.ShapeDtypeStruct((EE,), jnp.float32),
                  jax.ShapeDtypeStruct((NTILES, NN), jnp.float32)),
        mesh=_mesh(),
        scratch_types=[pltpu.VMEM((NN,), jnp.float32),
                       pltpu.VMEM((NN,), jnp.float32),
                       pltpu.VMEM((EPT,), jnp.int32),
                       pltpu.VMEM((EPT,), jnp.int32),
                       pltpu.VMEM((EPT,), jnp.float32),
                       pltpu.VMEM((NN,), jnp.float32),
                       pltpu.SemaphoreType.DMA],
        compiler_params=_SC_CP,
    )
    return kern(pq, src, dst)


def _scb_cfg(nchan):
    if nchan <= 40:
        return 4, 8, 1600
    return 4, 8, 800


def _make_scb_body(nchan):
    nrange, ngrp, chb = _scb_cfg(nchan)
    erange = EE // nrange
    nchunk = erange // chb
    nch_max = (nchan + ngrp - 1) // ngrp

    def body(h_hbm, src_hbm, dst_hbm, ee_hbm, raw_hbm, *scr):
        h_refs = scr[0:nch_max]
        o_refs = scr[nch_max:2 * nch_max]
        sbufs = scr[2 * nch_max:2 * nch_max + 2]
        dbufs = scr[2 * nch_max + 2:2 * nch_max + 4]
        ebufs = scr[2 * nch_max + 4:2 * nch_max + 6]
        sems = scr[2 * nch_max + 6:2 * nch_max + 8]

        core = lax.axis_index("c")
        sub = lax.axis_index("s")
        g = jnp.bitwise_and(sub, ngrp - 1)
        r = core * (nrange // 2) + lax.shift_right_logical(
            sub, ngrp.bit_length() - 1)
        ebase = r * erange

        for j in range(nch_max):
            c_eff = jnp.minimum(g + ngrp * j, nchan - 1)
            pltpu.sync_copy(h_hbm.at[c_eff], h_refs[j])

        @pl.loop(0, NN, step=16)
        def _(i):
            for j in range(nch_max):
                o_refs[j][pl.ds(i, 16)] = jnp.zeros((16,), jnp.float32)

        def start(chunk, b):
            off = ebase + chunk * chb
            pltpu.async_copy(src_hbm.at[pl.ds(off, chb)], sbufs[b], sems[b])
            pltpu.async_copy(dst_hbm.at[pl.ds(off, chb)], dbufs[b], sems[b])
            pltpu.async_copy(ee_hbm.at[pl.ds(off, chb)], ebufs[b], sems[b])

        def wait(b):
            pltpu.make_async_copy(src_hbm.at[pl.ds(0, chb)], sbufs[b],
                                  sems[b]).wait()
            pltpu.make_async_copy(dst_hbm.at[pl.ds(0, chb)], dbufs[b],
                                  sems[b]).wait()
            pltpu.make_async_copy(ee_hbm.at[pl.ds(0, chb)], ebufs[b],
                                  sems[b]).wait()

        def process(b):
            @plsc.parallel_loop(0, chb, step=16, unroll=4)
            def _(j):
                s16 = sbufs[b][pl.ds(j, 16)]
                d16 = dbufs[b][pl.ds(j, 16)]
                w16 = ebufs[b][pl.ds(j, 16)]
                for jch in range(nch_max):
                    gat = plsc.load_gather(h_refs[jch], [s16])
                    plsc.addupdate_scatter(o_refs[jch], [d16], gat * w16)

        start(0, 0)

        @pl.loop(0, nchunk, step=2)
        def _(i):
            start(i + 1, 1)
            wait(0)
            process(0)

            @pl.when(i + 2 < nchunk)
            def _():
                start(i + 2, 0)

            wait(1)
            process(1)

        for j in range(nch_max):
            c = g + ngrp * j

            @pl.when(c < nchan)
            def _():
                pltpu.sync_copy(o_refs[j], raw_hbm.at[r, c])

    return body


def _sc_phase_b(ht, src, dst, ee):
    nchan = ht.shape[0]
    nrange, ngrp, chb = _scb_cfg(nchan)
    nch_max = (nchan + ngrp - 1) // ngrp
    kern = pl.kernel(
        _make_scb_body(nchan),
        out_type=jax.ShapeDtypeStruct((nrange, nchan, NN), jnp.float32),
        mesh=_mesh(),
        scratch_types=([pltpu.VMEM((NN,), jnp.float32)] * (2 * nch_max)
                       + [pltpu.VMEM((chb,), jnp.int32)] * 4
                       + [pltpu.VMEM((chb,), jnp.float32)] * 2
                       + [pltpu.SemaphoreType.DMA] * 2),
        compiler_params=_SC_CP,
    )
    return kern(ht, src, dst, ee)


def _col(v):
    return v.reshape(-1, 1)


def _block_args(block):
    out = []
    for (g, b, w) in block:
        out += [_col(g), _col(b), w.T]
    return out


def kernel(x, edge_index, params):
    src = edge_index[0]
    dst = edge_index[1]
    mlp = params['mlp']

    w_g1, a_g1, bias_g1 = params['trans1']
    w_g2, a_g2, bias_g2 = params['trans2']
    c1 = w_g1.shape[0]
    o1 = w_g1.shape[1]
    c2 = w_g2.shape[0]
    o2 = w_g2.shape[1]

    xt = x.T

    tc1 = pl.pallas_call(
        _tc1_body,
        out_shape=(jax.ShapeDtypeStruct((o1, NN), jnp.float32),
                   jax.ShapeDtypeStruct((2, NN), jnp.float32)),
        compiler_params=_TC_CP,
    )
    ht1, pq1 = tc1(
        xt, mlp['W1'].T, _col(mlp['b1']), mlp['W2'].T, _col(mlp['b2']),
        mlp['W3'].T, _col(mlp['b3']),
        *_block_args(params['block1']),
        w_g1.T, a_g1[:o1].T, a_g1[o1:].T,
    )

    ee1, den1 = _sc_phase_a(pq1, src, dst)
    raw1 = _sc_phase_b(ht1, src, dst, ee1)

    tc2 = pl.pallas_call(
        _tc_mid_body,
        out_shape=(jax.ShapeDtypeStruct((o2, NN), jnp.float32),
                   jax.ShapeDtypeStruct((2, NN), jnp.float32)),
        compiler_params=_TC_CP,
    )
    ht2, pq2 = tc2(
        raw1, den1, _col(bias_g1),
        *_block_args(params['block2']),
        w_g2.T, a_g2[:o2].T, a_g2[o2:].T,
    )

    ee2, den2 = _sc_phase_a(pq2, src, dst)
    raw2 = _sc_phase_b(ht2, src, dst, ee2)

    gr = params['block3'][0][2].shape[1]
    tc3 = pl.pallas_call(
        _tc3_body,
        out_shape=(jax.ShapeDtypeStruct((o2, NN), jnp.float32),
                   jax.ShapeDtypeStruct((gr, NN), jnp.float32),
                   jax.ShapeDtypeStruct((gr, NN), jnp.float32),
                   jax.ShapeDtypeStruct((gr, NN), jnp.float32)),
        compiler_params=_TC_CP,
    )
    s0, s1, s2, s3 = tc3(
        raw2, den2, _col(bias_g2),
        *_block_args(params['block3']),
    )

    return jnp.concatenate([s0.T, s1.T, s2.T, s3.T], axis=1)


# speedup vs baseline: 49.1542x; 1.0022x over previous
import dataclasses
import jax
import jax.numpy as jnp
from jax import lax
from jax.experimental import pallas as pl
from jax.experimental.pallas import tpu as pltpu
from jax.experimental.pallas import tpu_sc as plsc

NN = 10000
EE = 320000
SLOPE_GAT = 0.2
NTILES = 32
EPT = EE // NTILES
CHB = 4000

_SC_CP = pltpu.CompilerParams()
if "needs_layout_passes" in pltpu.CompilerParams.__dataclass_fields__:
    _SC_CP = dataclasses.replace(_SC_CP, needs_layout_passes=False)
_TC_CP = pltpu.CompilerParams(vmem_limit_bytes=100 * 1024 * 1024)

_mesh_cache = []


def _mesh():
    if not _mesh_cache:
        _mesh_cache.append(
            plsc.VectorSubcoreMesh(core_axis_name="c", subcore_axis_name="s"))
    return _mesh_cache[0]

_PREC = lax.Precision.DEFAULT


def _leaky(x, s=0.01):
    return jnp.where(x >= 0, x, s * x)


def _mm(a, b):
    return lax.dot_general(a, b, (((1,), (0,)), ((), ())), precision=_PREC)


def _gdb_T(segs, layers):
    norms = [None] * len(segs)
    for (g, b, wt) in layers:
        off = 0
        y = None
        for j, s in enumerate(segs):
            cj = s.shape[0]
            if norms[j] is None:
                mu = jnp.mean(s, axis=1, keepdims=True)
                d = s - mu
                var = jnp.mean(d * d, axis=1, keepdims=True)
                norms[j] = d / jnp.sqrt(var + 1e-5)
            t = _leaky(g[off:off + cj] * norms[j] + b[off:off + cj])
            contrib = _mm(wt[:, off:off + cj], t)
            y = contrib if y is None else y + contrib
            off += cj
        segs = segs + [y]
        norms = norms + [None]
    return segs


def _gat_proj(segs, wt, a1t, a2t):
    off = 0
    ht = None
    for s in segs:
        cj = s.shape[0]
        contrib = _mm(wt[:, off:off + cj], s)
        ht = contrib if ht is None else ht + contrib
        off += cj
    p = _mm(a1t, ht)
    q = _mm(a2t, ht)
    pq = jnp.concatenate([p, q], axis=0)
    return ht, pq


def _tc1_body(x_t_ref, w1t, b1, w2t, b2, w3t, b3,
              g1, be1, wt1, g2, be2, wt2, g3, be3, wt3,
              wtg, a1t, a2t,
              ht_out, pq_out):
    xt = x_t_ref[...]
    h = _leaky(_mm(w1t[...], xt) + b1[...])
    h = _leaky(_mm(w2t[...], h) + b2[...])
    h = _leaky(_mm(w3t[...], h) + b3[...])
    segs = _gdb_T([h], [(g1[...], be1[...], wt1[...]),
                        (g2[...], be2[...], wt2[...]),
                        (g3[...], be3[...], wt3[...])])
    ht, pq = _gat_proj(segs, wtg[...], a1t[...], a2t[...])
    ht_out[...] = ht
    pq_out[...] = pq


def _tc_mid_body(raw_ref, den_ref, bias, g1, be1, wt1, g2, be2, wt2,
                 g3, be3, wt3, wtg, a1t, a2t, ht_out, pq_out):
    den = jnp.sum(den_ref[...], axis=0, keepdims=True)
    raw = jnp.sum(raw_ref[...], axis=0)
    x0 = raw / (den + 1e-16) + bias[...]
    segs = _gdb_T([x0], [(g1[...], be1[...], wt1[...]),
                         (g2[...], be2[...], wt2[...]),
                         (g3[...], be3[...], wt3[...])])
    ht, pq = _gat_proj(segs, wtg[...], a1t[...], a2t[...])
    ht_out[...] = ht
    pq_out[...] = pq


def _tc3_body(raw_ref, den_ref, bias, g1, be1, wt1, g2, be2, wt2,
              g3, be3, wt3, o0, o1, o2, o3):
    den = jnp.sum(den_ref[...], axis=0, keepdims=True)
    raw = jnp.sum(raw_ref[...], axis=0)
    x0 = raw / (den + 1e-16) + bias[...]
    segs = _gdb_T([x0], [(g1[...], be1[...], wt1[...]),
                         (g2[...], be2[...], wt2[...]),
                         (g3[...], be3[...], wt3[...])])
    o0[...] = segs[0]
    o1[...] = segs[1]
    o2[...] = segs[2]
    o3[...] = segs[3]


def _sca_body(pq_hbm, src_hbm, dst_hbm, ee_hbm, den_hbm,
              p_v, q_v, src_v, dst_v, ee_v, den_v, sem):
    wid = lax.axis_index("c") * 16 + lax.axis_index("s")
    base = wid * EPT
    pltpu.async_copy(pq_hbm.at[0], p_v, sem)
    pltpu.async_copy(pq_hbm.at[1], q_v, sem)
    pltpu.async_copy(src_hbm.at[pl.ds(base, EPT)], src_v, sem)
    pltpu.async_copy(dst_hbm.at[pl.ds(base, EPT)], dst_v, sem)
    pltpu.make_async_copy(pq_hbm.at[0], p_v, sem).wait()
    pltpu.make_async_copy(pq_hbm.at[1], q_v, sem).wait()
    pltpu.make_async_copy(src_hbm.at[pl.ds(0, EPT)], src_v, sem).wait()
    pltpu.make_async_copy(dst_hbm.at[pl.ds(0, EPT)], dst_v, sem).wait()

    @pl.loop(0, NN, step=16)
    def _(i):
        den_v[pl.ds(i, 16)] = jnp.zeros((16,), jnp.float32)

    @plsc.parallel_loop(0, EPT, step=16, unroll=8)
    def _(i):
        s16 = src_v[pl.ds(i, 16)]
        d16 = dst_v[pl.ds(i, 16)]
        e = plsc.load_gather(p_v, [d16]) + plsc.load_gather(q_v, [s16])
        e = jnp.where(e >= 0, e, SLOPE_GAT * e)
        ee = jnp.exp(e)
        ee_v[pl.ds(i, 16)] = ee
        plsc.addupdate_scatter(den_v, [d16], ee)

    pltpu.sync_copy(ee_v, ee_hbm.at[pl.ds(base, EPT)])
    pltpu.sync_copy(den_v, den_hbm.at[wid])


def _sc_phase_a(pq, src, dst):
    kern = pl.kernel(
        _sca_body,
        out_type=(jax.ShapeDtypeStruct((EE,), jnp.float32),
                  jax.ShapeDtypeStruct((NTILES, NN), jnp.float32)),
        mesh=_mesh(),
        scratch_types=[pltpu.VMEM((NN,), jnp.float32),
                       pltpu.VMEM((NN,), jnp.float32),
                       pltpu.VMEM((EPT,), jnp.int32),
                       pltpu.VMEM((EPT,), jnp.int32),
                       pltpu.VMEM((EPT,), jnp.float32),
                       pltpu.VMEM((NN,), jnp.float32),
                       pltpu.SemaphoreType.DMA],
        compiler_params=_SC_CP,
    )
    return kern(pq, src, dst)


def _scb_cfg(nchan):
    if nchan <= 40:
        return 4, 8, 1600
    return 4, 8, 800


def _make_scb_body(nchan):
    nrange, ngrp, chb = _scb_cfg(nchan)
    erange = EE // nrange
    nchunk = erange // chb
    nch_max = (nchan + ngrp - 1) // ngrp

    def body(h_hbm, src_hbm, dst_hbm, ee_hbm, raw_hbm, *scr):
        h_refs = scr[0:nch_max]
        o_refs = scr[nch_max:2 * nch_max]
        sbufs = scr[2 * nch_max:2 * nch_max + 2]
        dbufs = scr[2 * nch_max + 2:2 * nch_max + 4]
        ebufs = scr[2 * nch_max + 4:2 * nch_max + 6]
        sems = scr[2 * nch_max + 6:2 * nch_max + 8]

        core = lax.axis_index("c")
        sub = lax.axis_index("s")
        g = jnp.bitwise_and(sub, ngrp - 1)
        r = core * (nrange // 2) + lax.shift_right_logical(
            sub, ngrp.bit_length() - 1)
        ebase = r * erange

        for j in range(nch_max):
            c_eff = jnp.minimum(g + ngrp * j, nchan - 1)
            pltpu.sync_copy(h_hbm.at[c_eff], h_refs[j])

        @pl.loop(0, NN, step=16)
        def _(i):
            for j in range(nch_max):
                o_refs[j][pl.ds(i, 16)] = jnp.zeros((16,), jnp.float32)

        def start(chunk, b):
            off = ebase + chunk * chb
            pltpu.async_copy(src_hbm.at[pl.ds(off, chb)], sbufs[b], sems[b])
            pltpu.async_copy(dst_hbm.at[pl.ds(off, chb)], dbufs[b], sems[b])
            pltpu.async_copy(ee_hbm.at[pl.ds(off, chb)], ebufs[b], sems[b])

        def wait(b):
            pltpu.make_async_copy(src_hbm.at[pl.ds(0, chb)], sbufs[b],
                                  sems[b]).wait()
            pltpu.make_async_copy(dst_hbm.at[pl.ds(0, chb)], dbufs[b],
                                  sems[b]).wait()
            pltpu.make_async_copy(ee_hbm.at[pl.ds(0, chb)], ebufs[b],
                                  sems[b]).wait()

        def process(b):
            @plsc.parallel_loop(0, chb, step=16, unroll=4)
            def _(j):
                s16 = sbufs[b][pl.ds(j, 16)]
                d16 = dbufs[b][pl.ds(j, 16)]
                w16 = ebufs[b][pl.ds(j, 16)]
                for jch in range(nch_max):
                    gat = plsc.load_gather(h_refs[jch], [s16])
                    plsc.addupdate_scatter(o_refs[jch], [d16], gat * w16)

        start(0, 0)

        @pl.loop(0, nchunk, step=2)
        def _(i):
            start(i + 1, 1)
            wait(0)
            process(0)

            @pl.when(i + 2 < nchunk)
            def _():
                start(i + 2, 0)

            wait(1)
            process(1)

        for j in range(nch_max):
            c = g + ngrp * j

            @pl.when(c < nchan)
            def _():
                pltpu.sync_copy(o_refs[j], raw_hbm.at[r, c])

    return body


def _sc_phase_b(ht, src, dst, ee):
    nchan = ht.shape[0]
    nrange, ngrp, chb = _scb_cfg(nchan)
    nch_max = (nchan + ngrp - 1) // ngrp
    kern = pl.kernel(
        _make_scb_body(nchan),
        out_type=jax.ShapeDtypeStruct((nrange, nchan, NN), jnp.float32),
        mesh=_mesh(),
        scratch_types=([pltpu.VMEM((NN,), jnp.float32)] * (2 * nch_max)
                       + [pltpu.VMEM((chb,), jnp.int32)] * 4
                       + [pltpu.VMEM((chb,), jnp.float32)] * 2
                       + [pltpu.SemaphoreType.DMA] * 2),
        compiler_params=_SC_CP,
    )
    return kern(ht, src, dst, ee)


def _col(v):
    return v.reshape(-1, 1)


def _block_args(block):
    out = []
    for (g, b, w) in block:
        out += [_col(g), _col(b), w.T]
    return out


def kernel(x, edge_index, params):
    src = edge_index[0]
    dst = edge_index[1]
    mlp = params['mlp']

    w_g1, a_g1, bias_g1 = params['trans1']
    w_g2, a_g2, bias_g2 = params['trans2']
    c1 = w_g1.shape[0]
    o1 = w_g1.shape[1]
    c2 = w_g2.shape[0]
    o2 = w_g2.shape[1]

    xt = x.T

    tc1 = pl.pallas_call(
        _tc1_body,
        out_shape=(jax.ShapeDtypeStruct((o1, NN), jnp.float32),
                   jax.ShapeDtypeStruct((2, NN), jnp.float32)),
        compiler_params=_TC_CP,
    )
    ht1, pq1 = tc1(
        xt, mlp['W1'].T, _col(mlp['b1']), mlp['W2'].T, _col(mlp['b2']),
        mlp['W3'].T, _col(mlp['b3']),
        *_block_args(params['block1']),
        w_g1.T, a_g1[:o1].T, a_g1[o1:].T,
    )

    ee1, den1 = _sc_phase_a(pq1, src, dst)
    raw1 = _sc_phase_b(ht1, src, dst, ee1)

    tc2 = pl.pallas_call(
        _tc_mid_body,
        out_shape=(jax.ShapeDtypeStruct((o2, NN), jnp.float32),
                   jax.ShapeDtypeStruct((2, NN), jnp.float32)),
        compiler_params=_TC_CP,
    )
    ht2, pq2 = tc2(
        raw1, den1, _col(bias_g1),
        *_block_args(params['block2']),
        w_g2.T, a_g2[:o2].T, a_g2[o2:].T,
    )

    ee2, den2 = _sc_phase_a(pq2, src, dst)
    raw2 = _sc_phase_b(ht2, src, dst, ee2)

    gr = params['block3'][0][2].shape[1]
    tc3 = pl.pallas_call(
        _tc3_body,
        out_shape=(jax.ShapeDtypeStruct((o2, NN), jnp.float32),
                   jax.ShapeDtypeStruct((gr, NN), jnp.float32),
                   jax.ShapeDtypeStruct((gr, NN), jnp.float32),
                   jax.ShapeDtypeStruct((gr, NN), jnp.float32)),
        compiler_params=_TC_CP,
    )
    s0, s1, s2, s3 = tc3(
        raw2, den2, _col(bias_g2),
        *_block_args(params['block3']),
    )

    return jnp.concatenate([s0.T, s1.T, s2.T, s3.T], axis=1)
